# baseline (speedup 1.0000x reference)
import jax
import jax.numpy as jnp
from jax.experimental import pallas as pl
from jax.experimental.pallas import tpu as pltpu
from jax.experimental.pallas import tpu_sc as plsc

N_TOK = 8192
D_IN = 768
D_CODE = 32
K = 8192

BLK_A = 1024
BLK_C = 1024
GW = 256


def _encode_body(x_ref, we_ref, be_ref, ze_ref, zt_ref):
    z_e = jnp.dot(x_ref[...].astype(jnp.bfloat16),
                  we_ref[...].astype(jnp.bfloat16),
                  preferred_element_type=jnp.float32)
    z_e = z_e + be_ref[...]
    ze_ref[...] = z_e
    zt_ref[...] = z_e.astype(jnp.bfloat16).T


def _decode_body(ze_ref, zq_ref, wd_ref, bd_ref, zst_ref, xr_ref):
    z_e = ze_ref[...]
    z_q = zq_ref[:, :D_CODE]
    z_st = z_e + (z_q - z_e)
    zst_ref[...] = z_st
    xr = jnp.dot(z_st.astype(jnp.bfloat16),
                 wd_ref[...].astype(jnp.bfloat16),
                 preferred_element_type=jnp.float32)
    xr_ref[...] = xr + bd_ref[...]


def _sc_gather(cb_pad, ids_row):
    vector_mesh = plsc.VectorSubcoreMesh(core_axis_name="core",
                                         subcore_axis_name="subcore")

    @pl.kernel(out_type=jax.ShapeDtypeStruct((N_TOK, 128), cb_pad.dtype),
               mesh=vector_mesh)
    def gather_kernel(cb_hbm, i_hbm, o_hbm):
        def body(i_vmem, o_vmem):
            pltpu.sync_copy(cb_hbm.at[i_vmem.at[0]], o_vmem)

        pltpu.emit_pipeline(
            body,
            grid=(N_TOK // GW,),
            in_specs=[pl.BlockSpec((1, GW), index_map=lambda i: (0, i))],
            out_specs=[pl.BlockSpec((GW, 128), index_map=lambda i: (i, 0))],
            core_axis_name=("core", "subcore"),
            dimension_semantics=(pltpu.PARALLEL,),
        )(i_hbm, o_hbm)

    return gather_kernel(cb_pad, ids_row)


def kernel(x, W_enc, b_enc, codebook, W_dec, b_dec):
    B, T, _ = x.shape
    xf = x.reshape(N_TOK, D_IN)

    z_e_flat, z_e_t16 = pl.pallas_call(
        _encode_body,
        grid=(N_TOK // BLK_A,),
        in_specs=[
            pl.BlockSpec((BLK_A, D_IN), lambda i: (i, 0)),
            pl.BlockSpec((D_IN, D_CODE), lambda i: (0, 0)),
            pl.BlockSpec((1, D_CODE), lambda i: (0, 0)),
        ],
        out_specs=[
            pl.BlockSpec((BLK_A, D_CODE), lambda i: (i, 0)),
            pl.BlockSpec((D_CODE, BLK_A), lambda i: (0, i)),
        ],
        out_shape=[
            jax.ShapeDtypeStruct((N_TOK, D_CODE), jnp.float32),
            jax.ShapeDtypeStruct((D_CODE, N_TOK), jnp.bfloat16),
        ],
    )(xf, W_enc, b_enc[None, :])

    z_e = z_e_flat.reshape(B, T, D_CODE)
    flat = z_e.reshape(-1, D_CODE)
    fc = jax.lax.dot_general(z_e_t16.T, codebook,
                             (((1,), (1,)), ((), ())),
                             preferred_element_type=jnp.float32)
    dists = (jnp.sum(flat * flat, axis=1, keepdims=True)
             - 2.0 * fc
             + jnp.sum(codebook * codebook, axis=1)[None, :])
    ids = jnp.argmin(dists, axis=1)

    ids_row = ids.astype(jnp.int32).reshape(1, N_TOK)
    cb_pad = jnp.pad(codebook, ((0, 0), (0, 128 - D_CODE)))
    z_q = _sc_gather(cb_pad, ids_row)

    z_q_st, x_rec = pl.pallas_call(
        _decode_body,
        grid=(N_TOK // BLK_C,),
        in_specs=[
            pl.BlockSpec((BLK_C, D_CODE), lambda i: (i, 0)),
            pl.BlockSpec((BLK_C, 128), lambda i: (i, 0)),
            pl.BlockSpec((D_CODE, D_IN), lambda i: (0, 0)),
            pl.BlockSpec((1, D_IN), lambda i: (0, 0)),
        ],
        out_specs=[
            pl.BlockSpec((BLK_C, D_CODE), lambda i: (i, 0)),
            pl.BlockSpec((BLK_C, D_IN), lambda i: (i, 0)),
        ],
        out_shape=[
            jax.ShapeDtypeStruct((N_TOK, D_CODE), jnp.float32),
            jax.ShapeDtypeStruct((N_TOK, D_IN), jnp.float32),
        ],
    )(z_e_flat, z_q, W_dec, b_dec[None, :])

    return (x_rec.reshape(B, T, D_IN),
            z_e.reshape(B, T, D_CODE),
            z_q_st.reshape(B, T, D_CODE))
